# Initial kernel scaffold; baseline (speedup 1.0000x reference)
import jax
import jax.numpy as jnp
from jax.experimental import pallas as pl


def kernel(x, local_lgx, global_lgx, local_g, global_g, lg, src_ids, dst_ids, params):
    raise NotImplementedError("write your pallas kernel here")


# speedup vs baseline: 19.2357x; 19.2357x over previous
import functools
import math

import jax
import jax.numpy as jnp
from jax import lax
from jax.experimental import pallas as pl
from jax.experimental.pallas import tpu as pltpu
from jax.experimental.pallas import tpu_sc as plsc

N = 10000
E = 160000
ELG = 320000
D = 256
H = 8
DK = 32
NP = 10240
BR = 640

NC, NS = 2, 16
NW = NC * NS
CE = 4496
CN = 2560


def _make_gather(n_rows, n_pairs, table_rows):
    PW = n_rows // NW
    B = 200
    NB = PW // B
    assert PW % B == 0
    mesh = plsc.VectorSubcoreMesh(core_axis_name="c", subcore_axis_name="s")
    out_type = [jax.ShapeDtypeStruct((n_rows, D), jnp.float32)] * n_pairs
    scratch = [
        pltpu.VMEM((PW,), jnp.int32),
        pltpu.VMEM((B, D), jnp.float32),
        pltpu.VMEM((B, D), jnp.float32),
        pltpu.SemaphoreType.DMA,
        pltpu.SemaphoreType.DMA,
    ]

    @functools.partial(pl.kernel, out_type=out_type, mesh=mesh,
                       scratch_types=scratch)
    def gather_k(*refs):
        tabs = refs[:n_pairs]
        idxs = refs[n_pairs:2 * n_pairs]
        outs = refs[2 * n_pairs:3 * n_pairs]
        idx_v, buf0, buf1, sem0, sem1 = refs[3 * n_pairs:]
        wid = lax.axis_index("s") * NC + lax.axis_index("c")
        base = wid * PW
        for t, ix, o in zip(tabs, idxs, outs):
            pltpu.sync_copy(ix.at[pl.ds(base, PW)], idx_v)
            pltpu.async_copy(t.at[idx_v.at[pl.ds(0, B)]], buf0, sem0)

            def body2(jj, carry, t=t, o=o):
                j0 = jj * 2
                pltpu.make_async_copy(t.at[idx_v.at[pl.ds(0, B)]], buf0,
                                      sem0).wait()
                pltpu.async_copy(t.at[idx_v.at[pl.ds((j0 + 1) * B, B)]],
                                 buf1, sem1)
                pltpu.sync_copy(buf0, o.at[pl.ds(base + j0 * B, B)])
                pltpu.make_async_copy(t.at[idx_v.at[pl.ds(0, B)]], buf1,
                                      sem1).wait()

                @pl.when(j0 + 2 < NB)
                def _():
                    pltpu.async_copy(t.at[idx_v.at[pl.ds((j0 + 2) * B, B)]],
                                     buf0, sem0)

                pltpu.sync_copy(buf1, o.at[pl.ds(base + (j0 + 1) * B, B)])
                return carry

            lax.fori_loop(0, NB // 2, body2, 0)
            if NB % 2 == 1:
                j0 = NB - 1
                pltpu.make_async_copy(t.at[idx_v.at[pl.ds(0, B)]], buf0,
                                      sem0).wait()
                pltpu.sync_copy(buf0, o.at[pl.ds(base + j0 * B, B)])

    return gather_k


def _make_scatter(R, C, NCH):
    ACC = C + 16
    ZB = ACC // 8
    DB = C // 8
    NZI = (ZB + NS - 1) // NS
    NDI = (DB + NS - 1) // NS
    PW = R // NS
    NB = PW // 16
    assert C % 16 == 0 and ACC % 16 == 0 and PW % 16 == 0
    mesh = plsc.VectorSubcoreMesh(core_axis_name="c", subcore_axis_name="s")
    S_out = NCH * C
    out_type = [jax.ShapeDtypeStruct((S_out, D), jnp.float32),
                jax.ShapeDtypeStruct((S_out, 128), jnp.float32)]
    scratch = [
        pltpu.VMEM((PW,), jnp.int32),
        pltpu.VMEM((16,), jnp.int32),
        pltpu.VMEM((16,), jnp.int32),
        pltpu.VMEM((16, D), jnp.float32),
        pltpu.VMEM((16, D), jnp.float32),
        pltpu.VMEM((16, 128), jnp.float32),
        pltpu.VMEM((16, 128), jnp.float32),
        pltpu.SemaphoreType.DMA,
        pltpu.SemaphoreType.DMA,
        pltpu.SemaphoreType.DMA,
        pltpu.SemaphoreType.DMA,
        pltpu.VMEM_SHARED((ACC, D), jnp.float32),
        pltpu.VMEM_SHARED((ACC, 128), jnp.float32),
    ]

    @functools.partial(pl.kernel, out_type=out_type, mesh=mesh,
                       scratch_types=scratch)
    def scatter_k(rows_hbm, zrows_hbm, dst_hbm, zc_hbm, zc128_hbm,
                  outw, outz, dst_v, idx0_v, idx1_v, row0_v, row1_v,
                  zrow0_v, zrow1_v, semg0, semg1, sema0, sema1, accw, accz):
        core = lax.axis_index("c")
        tid = lax.axis_index("s")
        base_e = tid * PW
        pltpu.sync_copy(dst_hbm.at[pl.ds(base_e, PW)], dst_v)
        cu = jnp.full((16,), C, jnp.uint32)
        bufs = ((idx0_v, row0_v, zrow0_v, semg0, sema0),
                (idx1_v, row1_v, zrow1_v, semg1, sema1))

        def gather_start(b, p):
            _, row_v, zrow_v, semg, _ = bufs[p]
            pltpu.async_copy(rows_hbm.at[pl.ds(base_e + b * 16, 16)],
                             row_v, semg)
            pltpu.async_copy(zrows_hbm.at[pl.ds(base_e + b * 16, 16)],
                             zrow_v, semg)

        def accum(b, p, lo):
            idx_v, row_v, zrow_v, semg, sema = bufs[p]
            pltpu.make_async_copy(rows_hbm.at[pl.ds(base_e + b * 16, 16)],
                                  row_v, semg).wait()
            pltpu.make_async_copy(zrows_hbm.at[pl.ds(base_e + b * 16, 16)],
                                  zrow_v, semg).wait()
            d = dst_v[pl.ds(b * 16, 16)]
            loc = plsc.bitcast(
                jnp.minimum(plsc.bitcast(d - lo, jnp.uint32), cu), jnp.int32)
            idx_v[...] = loc
            pltpu.sync_copy(row_v, accw.at[idx_v], add=True)
            pltpu.sync_copy(zrow_v, accz.at[idx_v], add=True)

        def chunk_body(t, carry):
            ch = 2 * t + core
            lo = ch * C
            def zero_body(k, cz):
                blk = k * NS + tid

                @pl.when(blk < ZB)
                def _():
                    off = blk * 8
                    pltpu.sync_copy(zc_hbm.at[pl.ds(off, 8)],
                                    accw.at[pl.ds(off, 8)])
                    pltpu.sync_copy(zc128_hbm.at[pl.ds(off, 8)],
                                    accz.at[pl.ds(off, 8)])
                return cz

            lax.fori_loop(0, NZI, zero_body, jnp.int32(0))
            plsc.subcore_barrier()

            gather_start(0, 0)
            gather_start(1, 1)

            def body(b, carry2):
                p = lax.rem(b, 2)

                def run(p, b=b):
                    accum(b, p, lo)

                    @pl.when(b + 2 < NB)
                    def _():
                        gather_start(b + 2, p)

                lax.cond(p == 0, lambda: run(0), lambda: run(1))
                return carry2

            lax.fori_loop(0, NB, body, jnp.int32(0))
            plsc.subcore_barrier()

            def drain_body(k, cd):
                blk = k * NS + tid

                @pl.when(blk < DB)
                def _():
                    off = blk * 8
                    pltpu.sync_copy(accw.at[pl.ds(off, 8)],
                                    outw.at[pl.ds(lo + off, 8)])
                    pltpu.sync_copy(accz.at[pl.ds(off, 8)],
                                    outz.at[pl.ds(lo + off, 8)])
                return cd

            lax.fori_loop(0, NDI, drain_body, jnp.int32(0))
            plsc.subcore_barrier()
            return carry

        lax.fori_loop(0, NCH // 2, chunk_body, jnp.int32(0))

    return scatter_k


def _dot(a, b):
    return jnp.dot(a, b, preferred_element_type=jnp.float32)


def _qkv_call(x, wq, bq, wk, wv, addq=None, addv=None):
    m = x.shape[0]
    grid = (m // BR,)
    nin = 5 + (2 if addq is not None else 0)

    def body(*refs):
        if addq is not None:
            xr, wqr, bqr, wkr, wvr, aqr, avr, qo, ko, vo = refs
        else:
            xr, wqr, bqr, wkr, wvr, qo, ko, vo = refs
        xb = xr[...]
        q = _dot(xb, wqr[...]) + bqr[...]
        if addq is not None:
            q = q + aqr[...]
        qo[...] = q
        ko[...] = _dot(xb, wkr[...])
        v = _dot(xb, wvr[...])
        if addv is not None:
            v = v + avr[...]
        vo[...] = v

    row = pl.BlockSpec((BR, D), lambda i: (i, 0))
    full = pl.BlockSpec((D, D), lambda i: (0, 0))
    vec = pl.BlockSpec((1, D), lambda i: (0, 0))
    in_specs = [row, full, vec, full, full]
    args = [x, wq, bq.reshape(1, D), wk, wv]
    if addq is not None:
        in_specs += [row, row]
        args += [addq, addv]
    outs = pl.pallas_call(
        body,
        grid=grid,
        in_specs=in_specs,
        out_specs=[row, row, row],
        out_shape=[jax.ShapeDtypeStruct((m, D), jnp.float32)] * 3,
    )(*args)
    return outs


def _score_call(ke, qd, ve, ebias, sel, selt, p16):
    m = ke.shape[0]
    grid = (m // BR,)

    def body(*refs):
        if ebias is not None:
            ker, qdr, ver, er, selr, seltr, p16r, wvo, zo = refs
        else:
            ker, qdr, ver, selr, seltr, p16r, wvo, zo = refs
        k = ker[...]
        v = ver[...]
        if ebias is not None:
            eb = er[...]
            k = k + eb
            v = v + eb
        prod = k * qdr[...]
        score = _dot(prod, selr[...]) * (1.0 / math.sqrt(DK))
        sexp = jnp.exp(jnp.clip(score, -10.0, 10.0))
        zo[...] = _dot(sexp, p16r[...])
        wvo[...] = v * _dot(sexp, seltr[...])

    row = pl.BlockSpec((BR, D), lambda i: (i, 0))
    selspec = pl.BlockSpec((D, H), lambda i: (0, 0))
    seltspec = pl.BlockSpec((H, D), lambda i: (0, 0))
    p16spec = pl.BlockSpec((H, 128), lambda i: (0, 0))
    in_specs = [row, row, row]
    args = [ke, qd, ve]
    if ebias is not None:
        in_specs.append(row)
        args.append(ebias)
    in_specs += [selspec, seltspec, p16spec]
    args += [sel, selt, p16]
    return pl.pallas_call(
        body,
        grid=grid,
        in_specs=in_specs,
        out_specs=[row, pl.BlockSpec((BR, 128), lambda i: (i, 0))],
        out_shape=[jax.ShapeDtypeStruct((m, D), jnp.float32),
                   jax.ShapeDtypeStruct((m, 128), jnp.float32)],
    )(*args)


def _ln(x, g, b):
    mu = jnp.mean(x, axis=-1, keepdims=True)
    var = jnp.mean((x - mu) * (x - mu), axis=-1, keepdims=True)
    return (x - mu) / jnp.sqrt(var + 1e-5) * g + b


def _finish_call(wv, z, x0, selt16, wo, bo, g1, b1, w1, c1, w2, c2, g2, b2):
    m = wv.shape[0]
    grid = (m // BR,)

    def body(wvr, zr, x0r, selt16r, wor, bor, g1r, b1r, w1r, c1r, w2r, c2r,
             g2r, b2r, outr):
        zbig = _dot(zr[...], selt16r[...])
        o = wvr[...] / (zbig + 1e-12)
        a = x0r[...] + _dot(o, wor[...]) + bor[...]
        a = _ln(a, g1r[...], b1r[...])
        h = jnp.maximum(_dot(a, w1r[...]) + c1r[...], 0.0)
        f = a + _dot(h, w2r[...]) + c2r[...]
        outr[...] = _ln(f, g2r[...], b2r[...])

    row = pl.BlockSpec((BR, D), lambda i: (i, 0))
    z16 = pl.BlockSpec((BR, 128), lambda i: (i, 0))
    st16 = pl.BlockSpec((128, D), lambda i: (0, 0))
    full = pl.BlockSpec((D, D), lambda i: (0, 0))
    vec = pl.BlockSpec((1, D), lambda i: (0, 0))
    w1s = pl.BlockSpec((D, 4 * D), lambda i: (0, 0))
    c1s = pl.BlockSpec((1, 4 * D), lambda i: (0, 0))
    w2s = pl.BlockSpec((4 * D, D), lambda i: (0, 0))
    return pl.pallas_call(
        body,
        grid=grid,
        in_specs=[row, z16, row, st16, full, vec, vec, vec, w1s, c1s, w2s,
                  vec, vec, vec],
        out_specs=row,
        out_shape=jax.ShapeDtypeStruct((m, D), jnp.float32),
    )(wv, z, x0, selt16, wo, bo.reshape(1, D), g1.reshape(1, D),
      b1.reshape(1, D), w1, c1.reshape(1, 4 * D), w2, c2.reshape(1, D),
      g2.reshape(1, D), b2.reshape(1, D))


_gather5 = None
_gather3 = None


def _build():
    global _gather5, _gather3
    if _gather5 is None:
        _gather5 = _make_gather(E, 5, None)
        _gather3 = _make_gather(ELG, 3, None)


def kernel(x, local_lgx, global_lgx, local_g, global_g, lg, src_ids, dst_ids,
           params):
    _build()
    p = params
    f32 = jnp.float32

    lanes = jnp.arange(D) // DK
    sel = (lanes[:, None] == jnp.arange(H)[None, :]).astype(f32)
    selt = sel.T
    p128 = jnp.eye(H, 128, dtype=f32)
    sel128 = (jnp.arange(256)[None, :] // DK ==
              jnp.arange(128)[:, None]).astype(f32)

    xp = jnp.pad(x, ((0, NP - N), (0, 0)))

    qn, kn, vn = _qkv_call(xp, p['nWq'], p['nbq'], p['nWk'], p['nWv'])

    lsrc, ldst = local_g[0], local_g[1]
    srcx, dstx, qd, ke, ve = _gather5(x, x, qn, kn, vn,
                                      src_ids, dst_ids, ldst, lsrc, lsrc)

    qe, kee, vee = _qkv_call(local_lgx, p['eWq'], p['ebq'], p['eWk'], p['eWv'],
                             addq=srcx, addv=dstx)

    wv_rows_n, z_rows_n = _score_call(ke, qd, ve, local_lgx, sel, selt, p128)

    wv_np = jax.ops.segment_sum(wv_rows_n, ldst, num_segments=NP)
    z_np = jax.ops.segment_sum(z_rows_n, ldst, num_segments=NP)

    lgs, lgd = lg[0], lg[1]
    qg, kg, vg = _gather3(qe, kee, vee, lgd, lgs, lgs)

    wv_rows_e, z_rows_e = _score_call(kg, qg, vg, None, sel, selt, p128)

    wv_e = jax.ops.segment_sum(wv_rows_e, lgd, num_segments=E)
    z_e = jax.ops.segment_sum(z_rows_e, lgd, num_segments=E)

    out_x = _finish_call(wv_np, z_np, xp, sel128, p['nWo'], p['nbo'],
                         p['nlng'], p['nlnb'], p['nf_W1'], p['nf_b1'],
                         p['nf_W2'], p['nf_b2'], p['nf_lng'], p['nf_lnb'])
    out_x = out_x[:N]

    out_lgx = _finish_call(wv_e, z_e, local_lgx, sel128, p['eWo'], p['ebo'],
                           p['elng'], p['elnb'], p['ef_W1'], p['ef_b1'],
                           p['ef_W2'], p['ef_b2'], p['ef_lng'], p['ef_lnb'])
    return (out_x, out_lgx)
